# baseline (speedup 1.0000x reference)
import functools

import jax
import jax.numpy as jnp
from jax import lax
from jax.experimental import pallas as pl
from jax.experimental.pallas import tpu as pltpu
from jax.experimental.pallas import tpu_sc as plsc


_SAMPLES_PER_BLOCK = 8
_SC_LANES = 16


def _sc_gather_coeffs(t, table_a, table_b):
    n = t.shape[0]
    tt = table_a.shape[0]
    mesh = plsc.VectorSubcoreMesh(core_axis_name="c", subcore_axis_name="s")

    @functools.partial(
        pl.kernel,
        mesh=mesh,
        out_type=[
            jax.ShapeDtypeStruct((n,), jnp.float32),
            jax.ShapeDtypeStruct((n,), jnp.float32),
        ],
        scratch_types=[
            pltpu.VMEM((n,), jnp.int32),
            pltpu.VMEM((n,), jnp.float32),
            pltpu.VMEM((n,), jnp.float32),
            pltpu.SemaphoreType.DMA,
        ],
    )
    def gather_kernel(t_hbm, ta_hbm, tb_hbm, oa_hbm, ob_hbm, t_v, oa_v, ob_v, sem):
        cid = lax.axis_index("c")
        sid = lax.axis_index("s")

        @pl.when(jnp.logical_and(cid == 0, sid == 0))
        def _():
            pltpu.sync_copy(t_hbm, t_v)
            pltpu.async_copy(ta_hbm.at[t_v], oa_v, sem).wait()
            pltpu.async_copy(tb_hbm.at[t_v], ob_v, sem).wait()
            pltpu.sync_copy(oa_v, oa_hbm)
            pltpu.sync_copy(ob_v, ob_hbm)

    return gather_kernel(t, table_a, table_b)


def _add_noise_block(a_ref, b_ref, x0_ref, noise_ref, out_ref):
    i = pl.program_id(0)
    rows = x0_ref.shape[0] // _SAMPLES_PER_BLOCK
    for k in range(_SAMPLES_PER_BLOCK):
        a = a_ref[i * _SAMPLES_PER_BLOCK + k]
        b = b_ref[i * _SAMPLES_PER_BLOCK + k]
        sl = pl.ds(k * rows, rows)
        out_ref[sl, :] = a * x0_ref[sl, :] + b * noise_ref[sl, :]


def kernel(x0, noise, t, sqrt_alphas_cumprod, sqrt_one_minus_alphas_cumprod):
    n, c, h, w = x0.shape
    rows = c * h
    x2 = x0.reshape(n * rows, w)
    n2 = noise.reshape(n * rows, w)
    blk_rows = rows * _SAMPLES_PER_BLOCK

    a_vec, b_vec = _sc_gather_coeffs(
        t, sqrt_alphas_cumprod, sqrt_one_minus_alphas_cumprod
    )

    out = pl.pallas_call(
        _add_noise_block,
        grid_spec=pltpu.PrefetchScalarGridSpec(
            num_scalar_prefetch=2,
            grid=(n // _SAMPLES_PER_BLOCK,),
            in_specs=[
                pl.BlockSpec((blk_rows, w), lambda i, *_: (i, 0)),
                pl.BlockSpec((blk_rows, w), lambda i, *_: (i, 0)),
            ],
            out_specs=pl.BlockSpec((blk_rows, w), lambda i, *_: (i, 0)),
        ),
        out_shape=jax.ShapeDtypeStruct((n * rows, w), x0.dtype),
        compiler_params=pltpu.CompilerParams(
            dimension_semantics=("arbitrary",),
        ),
    )(a_vec, b_vec, x2, n2)
    return out.reshape(n, c, h, w)


# speedup vs baseline: 1.2190x; 1.2190x over previous
import jax
import jax.numpy as jnp
from jax.experimental import pallas as pl
from jax.experimental.pallas import tpu as pltpu


_LANES = 256
_SAMPLES_PER_BLOCK = 4


def _add_noise_block(t_ref, sa_ref, sb_ref, x0_ref, noise_ref, out_ref):
    i = pl.program_id(0)
    rows = x0_ref.shape[0] // _SAMPLES_PER_BLOCK
    for k in range(_SAMPLES_PER_BLOCK):
        tt = t_ref[i * _SAMPLES_PER_BLOCK + k]
        a = sa_ref[tt]
        b = sb_ref[tt]
        sl = pl.ds(k * rows, rows)
        out_ref[sl, :] = a * x0_ref[sl, :] + b * noise_ref[sl, :]


def kernel(x0, noise, t, sqrt_alphas_cumprod, sqrt_one_minus_alphas_cumprod):
    n, c, h, w = x0.shape
    rows = c * h * w // _LANES
    x2 = x0.reshape(n * rows, _LANES)
    n2 = noise.reshape(n * rows, _LANES)
    blk_rows = rows * _SAMPLES_PER_BLOCK

    out = pl.pallas_call(
        _add_noise_block,
        grid_spec=pltpu.PrefetchScalarGridSpec(
            num_scalar_prefetch=3,
            grid=(n // _SAMPLES_PER_BLOCK,),
            in_specs=[
                pl.BlockSpec((blk_rows, _LANES), lambda i, *_: (i, 0)),
                pl.BlockSpec((blk_rows, _LANES), lambda i, *_: (i, 0)),
            ],
            out_specs=pl.BlockSpec((blk_rows, _LANES), lambda i, *_: (i, 0)),
        ),
        out_shape=jax.ShapeDtypeStruct((n * rows, _LANES), x0.dtype),
        compiler_params=pltpu.CompilerParams(
            dimension_semantics=("parallel",),
        ),
    )(t, sqrt_alphas_cumprod, sqrt_one_minus_alphas_cumprod, x2, n2)
    return out.reshape(n, c, h, w)
